# baseline (speedup 1.0000x reference)
import functools

import jax
import jax.numpy as jnp
from jax import lax
from jax.experimental import pallas as pl
from jax.experimental.pallas import tpu as pltpu
from jax.experimental.pallas import tpu_sc as plsc

_BATCH = 16384
_DIM = 64
_NTAB = 4
_NC = 2
_NS = 16
_NW = _NC * _NS
_BPW = _BATCH // _NW
_CHUNK = 128
_NCHUNK = _BPW // _CHUNK
_NCHUNKS_TOTAL = _NTAB * _NCHUNK
_NBUF = 4
_LAG = 2


def _body(idx_hbm, user_t, recipe_t, ingredient_t, nutrition_t, out_hbm,
          idx_v, bufs, gsems, ssems):
    tables = (user_t, recipe_t, ingredient_t, nutrition_t)
    wid = lax.axis_index("s") * _NC + lax.axis_index("c")
    base = wid * _BPW

    pltpu.sync_copy(idx_hbm.at[wid], idx_v)

    chunks = [(c, j) for c in range(_NTAB) for j in range(_NCHUNK)]
    hg = [None] * _NCHUNKS_TOTAL
    hs = [None] * _NCHUNKS_TOTAL

    def fire_store(i):
        c, j = chunks[i]
        k = i % _NBUF
        hg[i].wait()
        hs[i] = pltpu.async_copy(
            bufs[k],
            out_hbm.at[pl.ds(base + j * _CHUNK, _CHUNK),
                       pl.ds(c * _DIM, _DIM)],
            ssems[k])

    for i, (c, j) in enumerate(chunks):
        k = i % _NBUF
        if i >= _NBUF:
            hs[i - _NBUF].wait()
        hg[i] = pltpu.async_copy(
            tables[c].at[idx_v.at[c * _NCHUNK + j]], bufs[k], gsems[k])
        if i >= _LAG:
            fire_store(i - _LAG)
    for i in range(_NCHUNKS_TOTAL - _LAG, _NCHUNKS_TOTAL):
        fire_store(i)
    for i in range(_NCHUNKS_TOTAL - _NBUF, _NCHUNKS_TOTAL):
        hs[i].wait()


def _sc_call(idx, user_t, recipe_t, ingredient_t, nutrition_t):
    def body(idx_hbm, ut, rt, it, nt, out_hbm, idx_v, b0, b1, b2, b3,
             g0, g1, g2, g3, s0, s1, s2, s3):
        _body(idx_hbm, ut, rt, it, nt, out_hbm, idx_v,
              (b0, b1, b2, b3), (g0, g1, g2, g3), (s0, s1, s2, s3))

    f = pl.kernel(
        body,
        out_type=jax.ShapeDtypeStruct((_BATCH, _NTAB * _DIM), jnp.float32),
        mesh=plsc.VectorSubcoreMesh(core_axis_name="c", subcore_axis_name="s"),
        scratch_types=[
            pltpu.VMEM((_NCHUNKS_TOTAL, _CHUNK), jnp.int32),
        ] + [pltpu.VMEM((_CHUNK, _DIM), jnp.float32)] * _NBUF
          + [pltpu.SemaphoreType.DMA] * (2 * _NBUF),
        compiler_params=pltpu.CompilerParams(use_tc_tiling_on_sc=False),
    )
    return f(idx, user_t, recipe_t, ingredient_t, nutrition_t)


def kernel(uid, rid, ing, nut, user_table, recipe_table, ingredient_table,
           nutrition_table):
    idx = jnp.stack([uid, rid, ing, nut], axis=0).astype(jnp.int32)
    idx = idx.reshape(_NTAB, _NW, _NCHUNK, _CHUNK)
    idx = idx.transpose(1, 0, 2, 3).reshape(_NW, _NCHUNKS_TOTAL, _CHUNK)
    return _sc_call(idx, user_table.astype(jnp.float32),
                    recipe_table.astype(jnp.float32),
                    ingredient_table.astype(jnp.float32),
                    nutrition_table.astype(jnp.float32))
